# baseline (speedup 1.0000x reference)
import functools

import jax
import jax.numpy as jnp
from jax import lax
from jax.experimental import pallas as pl
from jax.experimental.pallas import tpu as pltpu
from jax.experimental.pallas import tpu_sc as plsc

B = 128
G = 8192
GBITS = 13
NUM_MASKED = 4915
L = 16
CHUNKS = G // L
DIGIT_BITS = (11, 11, 8)
NB_MAX = 1 << max(DIGIT_BITS)
N_WORKERS = 32
ROWS_PER_TILE = B // N_WORKERS
UNROLL = 2

_mesh = plsc.VectorSubcoreMesh(core_axis_name="c", subcore_axis_name="s")

_scratch = [
    pltpu.VMEM((ROWS_PER_TILE, G), jnp.float32),
    pltpu.VMEM((ROWS_PER_TILE, G), jnp.int32),
    pltpu.VMEM((ROWS_PER_TILE, G), jnp.int32),
] + [pltpu.VMEM((NB_MAX,), jnp.int32) for _ in range(ROWS_PER_TILE)]


@functools.partial(
    pl.kernel,
    out_type=jax.ShapeDtypeStruct((B, G), jnp.int32),
    mesh=_mesh,
    scratch_types=_scratch,
    compiler_params=pltpu.CompilerParams(needs_layout_passes=False),
)
def _argsort_rows(noise_hbm, out_hbm, noise_v, buf_a, buf_b, *hist):
    core = lax.axis_index("c")
    sub = lax.axis_index("s")
    wid = sub * 2 + core
    base_row = wid * ROWS_PER_TILE
    iota = lax.iota(jnp.int32, L)
    NWAY = ROWS_PER_TILE

    cnt0, _ = plsc.scan_count(jnp.zeros((L,), jnp.int32))
    c0 = jnp.min(cnt0)

    pltpu.sync_copy(noise_hbm.at[pl.ds(base_row, ROWS_PER_TILE)], noise_v)

    ones = jnp.ones((L,), jnp.int32)
    zeros = jnp.zeros((L,), jnp.int32)

    for p in range(3):
        nbits = DIGIT_BITS[p]
        nb = 1 << nbits
        shift = sum(DIGIT_BITS[:p])
        src = buf_a if p == 1 else buf_b
        dst = buf_b if p == 1 else buf_a

        def load_chunk(q, c):
            if p == 0:
                kv = plsc.bitcast(noise_v[q, pl.ds(c * L, L)], jnp.int32)
                d = kv & (nb - 1)
                pay = lax.shift_left(
                    lax.shift_right_logical(kv, DIGIT_BITS[0]), GBITS
                ) | (c * L + iota)
            else:
                pay = src[q, pl.ds(c * L, L)]
                d = lax.shift_right_logical(pay, GBITS + shift - DIGIT_BITS[0])
                if p == 1:
                    d = d & (nb - 1)
                else:
                    pay = pay & (G - 1)
            return d, pay

        def clr(i, carry):
            for q in range(NWAY):
                hist[q][pl.ds(i * L, L)] = zeros
            return carry

        lax.fori_loop(0, nb // L, clr, jnp.int32(0), unroll=UNROLL)

        def histo(c, carry):
            ds = [load_chunk(q, c)[0] for q in range(NWAY)]
            for q in range(NWAY):
                plsc.addupdate_scatter(hist[q], [ds[q]], ones)
            return carry

        lax.fori_loop(0, CHUNKS, histo, jnp.int32(0), unroll=UNROLL)

        def scan(i, carry):
            nxt = []
            for q in range(NWAY):
                v = hist[q][pl.ds(i * L, L)]
                incl = plsc.cumsum(v)
                hist[q][pl.ds(i * L, L)] = incl - v + carry[q]
                nxt.append(carry[q] + jnp.max(incl))
            return tuple(nxt)

        lax.fori_loop(0, nb // L, scan, (jnp.int32(0),) * NWAY)

        def scat(c, carry):
            loaded = [load_chunk(q, c) for q in range(NWAY)]
            cnts = [plsc.scan_count(d)[0] for d, _ in loaded]
            for q in range(NWAY):
                d, pay = loaded[q]
                starts = plsc.load_gather(hist[q], [d])
                plsc.store_scatter(
                    dst, [jnp.full((L,), q, jnp.int32), starts + cnts[q] - c0],
                    pay)
                plsc.addupdate_scatter(hist[q], [d], ones)
            return carry

        lax.fori_loop(0, CHUNKS, scat, jnp.int32(0), unroll=UNROLL)

    pltpu.sync_copy(buf_a, out_hbm.at[pl.ds(base_row, ROWS_PER_TILE)])


def kernel(x, noise):
    del x
    perm = _argsort_rows(noise)
    return perm[:, :NUM_MASKED], perm[:, NUM_MASKED:]


# speedup vs baseline: 1.0896x; 1.0896x over previous
import functools

import jax
import jax.numpy as jnp
from jax import lax
from jax.experimental import pallas as pl
from jax.experimental.pallas import tpu as pltpu
from jax.experimental.pallas import tpu_sc as plsc

B = 128
G = 8192
GBITS = 13
NUM_MASKED = 4915
L = 16
CHUNKS = G // L
DIGIT_BITS = (11, 11, 8)
NB_MAX = 1 << max(DIGIT_BITS)
N_WORKERS = 32
ROWS_PER_TILE = B // N_WORKERS
UNROLL = 2

_mesh = plsc.VectorSubcoreMesh(core_axis_name="c", subcore_axis_name="s")

_scratch = []
for _ in range(ROWS_PER_TILE):
    _scratch += [
        pltpu.VMEM((G,), jnp.float32),
        pltpu.VMEM((G,), jnp.int32),
        pltpu.VMEM((G,), jnp.int32),
        pltpu.VMEM((NB_MAX,), jnp.int32),
    ]


@functools.partial(
    pl.kernel,
    out_type=jax.ShapeDtypeStruct((B, G), jnp.int32),
    mesh=_mesh,
    scratch_types=_scratch,
    compiler_params=pltpu.CompilerParams(needs_layout_passes=False),
)
def _argsort_rows(noise_hbm, out_hbm, *scratch):
    noise_v = scratch[0::4]
    buf_a = scratch[1::4]
    buf_b = scratch[2::4]
    hist = scratch[3::4]

    core = lax.axis_index("c")
    sub = lax.axis_index("s")
    wid = sub * 2 + core
    base_row = wid * ROWS_PER_TILE
    iota = lax.iota(jnp.int32, L)
    NWAY = ROWS_PER_TILE

    cnt0, _ = plsc.scan_count(jnp.zeros((L,), jnp.int32))
    c0 = jnp.min(cnt0)

    for q in range(NWAY):
        pltpu.sync_copy(noise_hbm.at[base_row + q], noise_v[q])

    ones = jnp.ones((L,), jnp.int32)
    zeros = jnp.zeros((L,), jnp.int32)

    for p in range(3):
        nbits = DIGIT_BITS[p]
        nb = 1 << nbits
        shift = sum(DIGIT_BITS[:p])
        src = buf_a if p == 1 else buf_b
        dst = buf_b if p == 1 else buf_a

        def load_chunk(q, c):
            if p == 0:
                kv = plsc.bitcast(noise_v[q][pl.ds(c * L, L)], jnp.int32)
                d = kv & (nb - 1)
                pay = lax.shift_left(
                    lax.shift_right_logical(kv, DIGIT_BITS[0]), GBITS
                ) | (c * L + iota)
            else:
                pay = src[q][pl.ds(c * L, L)]
                d = lax.shift_right_logical(pay, GBITS + shift - DIGIT_BITS[0])
                if p == 1:
                    d = d & (nb - 1)
                else:
                    pay = pay & (G - 1)
            return d, pay

        def clr(i, carry):
            for q in range(NWAY):
                hist[q][pl.ds(i * L, L)] = zeros
            return carry

        with jax.named_scope(f"clr{p}"):
            lax.fori_loop(0, nb // L, clr, jnp.int32(0), unroll=UNROLL)

        def histo(c, carry):
            ds = [load_chunk(q, c)[0] for q in range(NWAY)]
            for q in range(NWAY):
                plsc.addupdate_scatter(hist[q], [ds[q]], ones)
            return carry

        with jax.named_scope(f"histo{p}"):
            lax.fori_loop(0, CHUNKS, histo, jnp.int32(0), unroll=UNROLL)

        def scan(i, carry):
            nxt = []
            for q in range(NWAY):
                v = hist[q][pl.ds(i * L, L)]
                incl = plsc.cumsum(v)
                hist[q][pl.ds(i * L, L)] = incl - v + carry[q]
                nxt.append(carry[q] + jnp.max(incl))
            return tuple(nxt)

        with jax.named_scope(f"scan{p}"):
            lax.fori_loop(0, nb // L, scan, (jnp.int32(0),) * NWAY)

        def scat(c, carry):
            loaded = [load_chunk(q, c) for q in range(NWAY)]
            cnts = [plsc.scan_count(d)[0] for d, _ in loaded]
            for q in range(NWAY):
                d, pay = loaded[q]
                starts = plsc.load_gather(hist[q], [d])
                plsc.store_scatter(dst[q], [starts + cnts[q] - c0], pay)
                plsc.addupdate_scatter(hist[q], [d], ones)
            return carry

        with jax.named_scope(f"scat{p}"):
            lax.fori_loop(0, CHUNKS, scat, jnp.int32(0), unroll=UNROLL)

    for q in range(NWAY):
        pltpu.sync_copy(buf_a[q], out_hbm.at[base_row + q])


def kernel(x, noise):
    del x
    perm = _argsort_rows(noise)
    return perm[:, :NUM_MASKED], perm[:, NUM_MASKED:]


# speedup vs baseline: 1.1694x; 1.0732x over previous
import functools

import jax
import jax.numpy as jnp
from jax import lax
from jax.experimental import pallas as pl
from jax.experimental.pallas import tpu as pltpu
from jax.experimental.pallas import tpu_sc as plsc

B = 128
G = 8192
GBITS = 13
NUM_MASKED = 4915
L = 16
CHUNKS = G // L
D0, D1, D2 = 11, 11, 8
NB0, NB1, NB2 = 1 << D0, 1 << D1, 1 << D2
N_WORKERS = 32
ROWS_PER_TILE = B // N_WORKERS
UNROLL = 4

_mesh = plsc.VectorSubcoreMesh(core_axis_name="c", subcore_axis_name="s")

_scratch = []
for _ in range(ROWS_PER_TILE):
    _scratch += [
        pltpu.VMEM((G,), jnp.float32),
        pltpu.VMEM((G,), jnp.int32),
        pltpu.VMEM((G,), jnp.int32),
        pltpu.VMEM((NB0,), jnp.int32),
        pltpu.VMEM((NB1,), jnp.int32),
    ]


@functools.partial(
    pl.kernel,
    out_type=jax.ShapeDtypeStruct((B, G), jnp.int32),
    mesh=_mesh,
    scratch_types=_scratch,
    compiler_params=pltpu.CompilerParams(needs_layout_passes=False),
)
def _argsort_rows(noise_hbm, out_hbm, *scratch):
    noise_v = scratch[0::5]
    buf_a = scratch[1::5]
    buf_b = scratch[2::5]
    hist_a = scratch[3::5]
    hist_b = scratch[4::5]

    core = lax.axis_index("c")
    sub = lax.axis_index("s")
    wid = sub * 2 + core
    base_row = wid * ROWS_PER_TILE
    iota = lax.iota(jnp.int32, L)
    NWAY = ROWS_PER_TILE

    cnt0, _ = plsc.scan_count(jnp.zeros((L,), jnp.int32))
    c0 = jnp.min(cnt0)

    for q in range(NWAY):
        pltpu.sync_copy(noise_hbm.at[base_row + q], noise_v[q])

    ones = jnp.ones((L,), jnp.int32)
    zeros = jnp.zeros((L,), jnp.int32)

    def clear(refs, n):
        def clr(i, carry):
            for ref in refs:
                ref[pl.ds(i * L, L)] = zeros
            return carry
        lax.fori_loop(0, n // L, clr, jnp.int32(0), unroll=UNROLL)

    def excl_scan(refs, n, name):
        def scan(i, carry):
            nxt = []
            for k, ref in enumerate(refs):
                v = ref[pl.ds(i * L, L)]
                incl = plsc.cumsum(v)
                ref[pl.ds(i * L, L)] = incl - v + carry[k]
                nxt.append(carry[k] + jnp.max(incl))
            return tuple(nxt)
        with jax.named_scope(name):
            lax.fori_loop(0, n // L, scan, (jnp.int32(0),) * len(refs))

    with jax.named_scope("clr0"):
        clear(hist_a, NB0)

    def histo(c, carry):
        kvs = [plsc.bitcast(noise_v[q][pl.ds(c * L, L)], jnp.int32)
               for q in range(NWAY)]
        for q in range(NWAY):
            plsc.addupdate_scatter(hist_a[q], [kvs[q] & (NB0 - 1)], ones)
        return carry

    with jax.named_scope("histo0"):
        lax.fori_loop(0, CHUNKS, histo, jnp.int32(0), unroll=UNROLL)

    excl_scan(hist_a, NB0, "scan0")
    with jax.named_scope("clrB"):
        clear(hist_b, NB1)

    def scat0(c, carry):
        loaded = []
        for q in range(NWAY):
            kv = plsc.bitcast(noise_v[q][pl.ds(c * L, L)], jnp.int32)
            d = kv & (NB0 - 1)
            pay = lax.shift_left(lax.shift_right_logical(kv, D0), GBITS) \
                | (c * L + iota)
            loaded.append((d, pay))
        cnts = [plsc.scan_count(d)[0] for d, _ in loaded]
        for q in range(NWAY):
            d, pay = loaded[q]
            starts = plsc.load_gather(hist_a[q], [d])
            plsc.store_scatter(buf_a[q], [starts + cnts[q] - c0], pay)
            plsc.addupdate_scatter(hist_a[q], [d], ones)
            d1 = lax.shift_right_logical(pay, GBITS) & (NB1 - 1)
            plsc.addupdate_scatter(hist_b[q], [d1], ones)
        return carry

    with jax.named_scope("scat0"):
        lax.fori_loop(0, CHUNKS, scat0, jnp.int32(0), unroll=UNROLL)

    excl_scan(hist_b, NB1, "scan1")
    with jax.named_scope("clrA2"):
        clear(hist_a, NB2)

    def scat1(c, carry):
        pays = [buf_a[q][pl.ds(c * L, L)] for q in range(NWAY)]
        ds = [lax.shift_right_logical(pay, GBITS) & (NB1 - 1) for pay in pays]
        cnts = [plsc.scan_count(d)[0] for d in ds]
        for q in range(NWAY):
            starts = plsc.load_gather(hist_b[q], [ds[q]])
            plsc.store_scatter(buf_b[q], [starts + cnts[q] - c0], pays[q])
            plsc.addupdate_scatter(hist_b[q], [ds[q]], ones)
            d2 = lax.shift_right_logical(pays[q], GBITS + D1)
            plsc.addupdate_scatter(hist_a[q], [d2], ones)
        return carry

    with jax.named_scope("scat1"):
        lax.fori_loop(0, CHUNKS, scat1, jnp.int32(0), unroll=UNROLL)

    excl_scan(hist_a, NB2, "scan2")

    def scat2(c, carry):
        pays = [buf_b[q][pl.ds(c * L, L)] for q in range(NWAY)]
        ds = [lax.shift_right_logical(pay, GBITS + D1) for pay in pays]
        cnts = [plsc.scan_count(d)[0] for d in ds]
        for q in range(NWAY):
            starts = plsc.load_gather(hist_a[q], [ds[q]])
            plsc.store_scatter(buf_a[q], [starts + cnts[q] - c0],
                               pays[q] & (G - 1))
            plsc.addupdate_scatter(hist_a[q], [ds[q]], ones)
        return carry

    with jax.named_scope("scat2"):
        lax.fori_loop(0, CHUNKS, scat2, jnp.int32(0), unroll=UNROLL)

    for q in range(NWAY):
        pltpu.sync_copy(buf_a[q], out_hbm.at[base_row + q])


def kernel(x, noise):
    del x
    perm = _argsort_rows(noise)
    return perm[:, :NUM_MASKED], perm[:, NUM_MASKED:]
